# baseline (speedup 1.0000x reference)
import functools

import jax
import jax.numpy as jnp
from jax import lax
from jax.experimental import pallas as pl
from jax.experimental.pallas import tpu as pltpu
from jax.experimental.pallas import tpu_sc as plsc

_N = 100000
_D = 512
_NC = 2
_NS = 16
_NW = _NC * _NS
_R = 40
_CHUNKS = _N // _R
_KMAX = -(-_CHUNKS // _NW)
_G = _D // 16
_RU = 8


def _make_sc_swap():
    mesh = plsc.VectorSubcoreMesh(core_axis_name="c", subcore_axis_name="s")

    @functools.partial(
        pl.kernel,
        mesh=mesh,
        out_type=[
            jax.ShapeDtypeStruct((_N, _D), jnp.float32),
            jax.ShapeDtypeStruct((_N, _D), jnp.float32),
        ],
        scratch_types=[
            pltpu.VMEM((_D,), jnp.int32),
            pltpu.VMEM((_R, _D), jnp.float32),
            pltpu.VMEM((_R, _D), jnp.float32),
            pltpu.VMEM((_R, _D), jnp.float32),
            pltpu.VMEM((_R, _D), jnp.float32),
        ],
    )
    def sc_swap(mask_hbm, x_hbm, xt_hbm, u_hbm, ut_hbm,
                mask_v, x_v, xt_v, u_v, ut_v):
        wid = lax.axis_index("s") * _NC + lax.axis_index("c")
        pltpu.sync_copy(mask_hbm, mask_v)

        def chunk(k, carry):
            ci = wid + k * _NW

            @pl.when(ci < _CHUNKS)
            def _():
                row0 = ci * _R
                pltpu.sync_copy(x_hbm.at[pl.ds(row0, _R)], x_v)
                pltpu.sync_copy(xt_hbm.at[pl.ds(row0, _R)], xt_v)
                for g in range(_G):
                    mb = mask_v[pl.ds(g * 16, 16)] != 0

                    def rows(b, c, mb=mb, g=g):
                        for j in range(_RU):
                            r = b * _RU + j
                            xv = x_v[r, pl.ds(g * 16, 16)]
                            tv = xt_v[r, pl.ds(g * 16, 16)]
                            u_v[r, pl.ds(g * 16, 16)] = jnp.where(mb, tv, xv)
                            ut_v[r, pl.ds(g * 16, 16)] = jnp.where(mb, xv, tv)
                        return c

                    lax.fori_loop(0, _R // _RU, rows, 0)
                pltpu.sync_copy(u_v, u_hbm.at[pl.ds(row0, _R)])
                pltpu.sync_copy(ut_v, ut_hbm.at[pl.ds(row0, _R)])

            return carry

        lax.fori_loop(0, _KMAX, chunk, 0)

    return sc_swap


_sc_swap = _make_sc_swap()


@jax.jit
def kernel(x, x_tilde):
    n, d = x.shape
    bool_swap = jax.random.bernoulli(jax.random.key(42), 0.5, (d,))
    mask_i = bool_swap.astype(jnp.int32)
    u, ut = _sc_swap(mask_i, x, x_tilde)
    return (u, ut)
